# baseline (speedup 1.0000x reference)
import functools
import jax, jax.numpy as jnp
from jax import lax
from jax.experimental import pallas as pl
from jax.experimental.pallas import tpu as pltpu
from jax.experimental.pallas import tpu_sc as plsc

def _body(b_ref, o_ref):
    o_ref[...] = jnp.broadcast_to(b_ref[...], o_ref.shape)

def kernel(hidden, tag, is_train, tag_table, W, b):
    B, T = 16384, 1000
    HALF = B // 2
    BT = 4096
    tc_part = pl.pallas_call(
        _body,
        grid=(HALF // BT,),
        in_specs=[pl.BlockSpec((1, T), lambda i: (0, 0))],
        out_specs=pl.BlockSpec((BT, T), lambda i: (i, 0)),
        out_shape=jax.ShapeDtypeStruct((HALF, T), jnp.float32),
        compiler_params=pltpu.CompilerParams(dimension_semantics=("arbitrary",)),
    )(b.reshape(1, T))

    info = plsc.get_sparse_core_info()
    nc, ns = info.num_cores, info.num_subcores
    nw = nc * ns
    rows_per_w = HALF // nw
    CH = 64
    mesh = plsc.VectorSubcoreMesh(core_axis_name="c", subcore_axis_name="s")

    @functools.partial(
        pl.kernel, mesh=mesh,
        out_type=jax.ShapeDtypeStruct((HALF, T), jnp.float32),
        scratch_types=[pltpu.VMEM((CH, T), jnp.float32)],
    )
    def wr(out_hbm, buf):
        wid = lax.axis_index("s") * nc + lax.axis_index("c")
        base = wid * rows_per_w
        for j in range(rows_per_w // CH):
            pltpu.sync_copy(buf, out_hbm.at[pl.ds(base + j * CH, CH)])

    sc_part = wr()
    return jnp.concatenate([tc_part, sc_part], axis=0)
